# Initial kernel scaffold; baseline (speedup 1.0000x reference)
import jax
import jax.numpy as jnp
from jax.experimental import pallas as pl


def kernel(x, table, kernel, bias):
    raise NotImplementedError("write your pallas kernel here")


# speedup vs baseline: 1.8954x; 1.8954x over previous
import functools

import jax
import jax.numpy as jnp
from jax import lax
from jax.experimental import pallas as pl
from jax.experimental.pallas import tpu as pltpu
from jax.experimental.pallas import tpu_sc as plsc

_VOCAB = 10000
_EMBED = 128
_BATCH = 16384

_info = plsc.get_sparse_core_info()
_NC = _info.num_cores
_NS = _info.num_subcores
_NW = _NC * _NS
_CHUNK = 128
_NCH = _BATCH // (_NW * _CHUNK)


def _tc_precompute_body(table_ref, w_ref, b_ref, y_ref):
    t = jnp.maximum(table_ref[...], 0.0)
    w = w_ref[...]
    acc = jnp.sum(t * w[:, 0][None, :], axis=1, keepdims=True)
    y_ref[...] = jax.nn.sigmoid(acc + b_ref[0, 0])


def _tc_precompute(table, w, b):
    return pl.pallas_call(
        _tc_precompute_body,
        out_shape=jax.ShapeDtypeStruct((_VOCAB, 1), jnp.float32),
    )(table, w, b)


_sc_mesh = plsc.VectorSubcoreMesh(core_axis_name="c", subcore_axis_name="s")


@functools.partial(
    pl.kernel,
    mesh=_sc_mesh,
    out_type=jax.ShapeDtypeStruct((_NW, _NCH, _CHUNK), jnp.float32),
    scratch_types=[
        pltpu.VMEM((_NCH, _CHUNK), jnp.int32),
        pltpu.VMEM((_NCH, _CHUNK), jnp.float32),
        pltpu.SemaphoreType.DMA,
    ],
)
def _sc_gather(idx_hbm, y_hbm, out_hbm, idx_v, vals_v, sem):
    wid = lax.axis_index("s") * _NC + lax.axis_index("c")
    pltpu.sync_copy(idx_hbm.at[wid], idx_v)
    copies = [
        pltpu.async_copy(y_hbm.at[idx_v.at[j]], vals_v.at[j], sem)
        for j in range(_NCH)
    ]
    for c in copies:
        c.wait()
    pltpu.sync_copy(vals_v, out_hbm.at[wid])


def kernel(x, table, kernel, bias):
    y = _tc_precompute(table, kernel, bias).reshape(_VOCAB)
    idx = x.astype(jnp.int32).reshape(_NW, _NCH, _CHUNK)
    out = _sc_gather(idx, y)
    return out.reshape(_BATCH, 1)


# speedup vs baseline: 1.9959x; 1.0530x over previous
import functools

import jax
import jax.numpy as jnp
from jax import lax
from jax.experimental import pallas as pl
from jax.experimental.pallas import tpu as pltpu
from jax.experimental.pallas import tpu_sc as plsc

_VOCAB = 10000
_EMBED = 128
_BATCH = 16384

_NC = 2
_NS = 16
_NW = _NC * _NS
_CHUNK = 128
_NCH = _BATCH // (_NW * _CHUNK)
_BPW = _NCH * _CHUNK


_TC_GRID = 10
_ROWS = 1024
_VPAD = _TC_GRID * _ROWS


def _tc_precompute_body(table_ref, w_ref, b_ref, y_ref):
    i = pl.program_id(0)
    t = jnp.maximum(table_ref[...], 0.0)
    w = w_ref[...]
    acc = jnp.sum(t * w[:, 0][None, :], axis=1)
    y_ref[pl.ds(i * _ROWS, _ROWS)] = jax.nn.sigmoid(acc + b_ref[0, 0])


def _tc_precompute(table, w, b):
    return pl.pallas_call(
        _tc_precompute_body,
        grid=(_TC_GRID,),
        in_specs=[
            pl.BlockSpec((_ROWS, _EMBED), lambda i: (i, 0)),
            pl.BlockSpec((_EMBED, 1), lambda i: (0, 0)),
            pl.BlockSpec((1, 1), lambda i: (0, 0)),
        ],
        out_specs=pl.BlockSpec((_VPAD,), lambda i: (0,)),
        out_shape=jax.ShapeDtypeStruct((_VPAD,), jnp.float32),
    )(table, w, b)


_sc_mesh = plsc.VectorSubcoreMesh(
    core_axis_name="c", subcore_axis_name="s", num_cores=_NC
)


@functools.partial(
    pl.kernel,
    mesh=_sc_mesh,
    out_type=jax.ShapeDtypeStruct((_BATCH,), jnp.float32),
    scratch_types=[
        pltpu.VMEM((_BPW,), jnp.int32),
        pltpu.VMEM((_BPW,), jnp.float32),
        pltpu.SemaphoreType.DMA,
    ],
)
def _sc_gather(idx_hbm, y_hbm, out_hbm, idx_v, vals_v, sem):
    wid = lax.axis_index("s") * _NC + lax.axis_index("c")
    base = wid * _BPW
    pltpu.sync_copy(idx_hbm.at[pl.ds(base, _BPW)], idx_v)
    copies = [
        pltpu.async_copy(
            y_hbm.at[idx_v.at[pl.ds(j * _CHUNK, _CHUNK)]],
            vals_v.at[pl.ds(j * _CHUNK, _CHUNK)],
            sem,
        )
        for j in range(_NCH)
    ]
    for c in copies:
        c.wait()
    pltpu.sync_copy(vals_v, out_hbm.at[pl.ds(base, _BPW)])


def kernel(x, table, kernel, bias):
    y = _tc_precompute(table, kernel, bias)
    return _sc_gather(x.astype(jnp.int32), y).reshape(_BATCH, 1)


# speedup vs baseline: 2.3679x; 1.1864x over previous
import functools

import jax
import jax.numpy as jnp
from jax import lax
from jax.experimental import pallas as pl
from jax.experimental.pallas import tpu as pltpu
from jax.experimental.pallas import tpu_sc as plsc

_VOCAB = 10000
_EMBED = 128
_BATCH = 16384

_NC = 2
_NS = 16
_NW = _NC * _NS
_CHUNK = 128
_NCH = _BATCH // (_NW * _CHUNK)
_BPW = _NCH * _CHUNK

_TC_CHUNKS = [(i * 1024, 1024) for i in range(9)] + [(9216, 784)]
_NBUF = 4


def _tc_precompute_body(table_hbm, w_hbm, b_ref, y_ref,
                        buf0, buf1, buf2, buf3, wv, sems):
    bufs = (buf0, buf1, buf2, buf3)
    wcopy = pltpu.make_async_copy(w_hbm, wv, sems.at[_NBUF])
    wcopy.start()
    copies = []
    for k, (off, sz) in enumerate(_TC_CHUNKS):
        copies.append(pltpu.make_async_copy(
            table_hbm.at[pl.ds(off, sz), :],
            bufs[k % _NBUF].at[pl.ds(0, sz), :],
            sems.at[k % _NBUF],
        ))
    for k in range(_NBUF):
        copies[k].start()
    wcopy.wait()
    w = wv[...]
    b = b_ref[0, 0]
    for k, (off, sz) in enumerate(_TC_CHUNKS):
        copies[k].wait()
        t = jnp.maximum(bufs[k % _NBUF][pl.ds(0, sz), :], 0.0)
        acc = jnp.dot(t, w, preferred_element_type=jnp.float32)
        yv = jnp.transpose(acc)
        y_ref[:, pl.ds(off, sz)] = jax.nn.sigmoid(yv + b)
        if k + _NBUF < len(_TC_CHUNKS):
            copies[k + _NBUF].start()


def _tc_precompute(table, w, b):
    table = pltpu.with_memory_space_constraint(table, pltpu.MemorySpace.HBM)
    w = pltpu.with_memory_space_constraint(w, pltpu.MemorySpace.HBM)
    return pl.pallas_call(
        _tc_precompute_body,
        in_specs=[
            pl.BlockSpec(memory_space=pl.ANY),
            pl.BlockSpec(memory_space=pl.ANY),
            pl.BlockSpec((1, 1), lambda: (0, 0)),
        ],
        out_shape=jax.ShapeDtypeStruct((1, _VOCAB), jnp.float32),
        scratch_shapes=[
            pltpu.VMEM((1024, _EMBED), jnp.float32),
            pltpu.VMEM((1024, _EMBED), jnp.float32),
            pltpu.VMEM((1024, _EMBED), jnp.float32),
            pltpu.VMEM((1024, _EMBED), jnp.float32),
            pltpu.VMEM((_EMBED, 1), jnp.float32),
            pltpu.SemaphoreType.DMA((_NBUF + 1,)),
        ],
    )(table, w, b)


_sc_mesh = plsc.VectorSubcoreMesh(
    core_axis_name="c", subcore_axis_name="s", num_cores=_NC
)


@functools.partial(
    pl.kernel,
    mesh=_sc_mesh,
    out_type=jax.ShapeDtypeStruct((_BATCH,), jnp.float32),
    scratch_types=[
        pltpu.VMEM((_BPW,), jnp.int32),
        pltpu.VMEM((_BPW,), jnp.float32),
        pltpu.SemaphoreType.DMA,
    ],
)
def _sc_gather(idx_hbm, y_hbm, out_hbm, idx_v, vals_v, sem):
    wid = lax.axis_index("s") * _NC + lax.axis_index("c")
    base = wid * _BPW
    pltpu.sync_copy(idx_hbm.at[pl.ds(base, _BPW)], idx_v)
    copies = [
        pltpu.async_copy(
            y_hbm.at[0].at[idx_v.at[pl.ds(j * _CHUNK, _CHUNK)]],
            vals_v.at[pl.ds(j * _CHUNK, _CHUNK)],
            sem,
        )
        for j in range(_NCH)
    ]
    for c in copies:
        c.wait()
    pltpu.sync_copy(vals_v, out_hbm.at[pl.ds(base, _BPW)])


def kernel(x, table, kernel, bias):
    y = _tc_precompute(table, kernel, bias)
    return _sc_gather(x.astype(jnp.int32), y).reshape(_BATCH, 1)
